# Initial kernel scaffold; baseline (speedup 1.0000x reference)
import jax
import jax.numpy as jnp
from jax.experimental import pallas as pl


def kernel(t, x, W, b):
    raise NotImplementedError("write your pallas kernel here")


# speedup vs baseline: 2.1889x; 2.1889x over previous
import functools

import jax
import jax.numpy as jnp
from jax.experimental import pallas as pl

_N = 50000
_H = 128
_GP_ROWS = 400


def _odefunc_body(g_ref, si_ref, x3_ref, wt_ref, b_ref, out_ref, *, block_rows):
    i = pl.program_id(0)
    B = block_rows
    k = jnp.sum((g_ref[...] != 0.0).astype(jnp.int32))
    v = si_ref[...].reshape(2 * B, _H)
    sir = jax.nn.sigmoid(
        jax.lax.dot_general(
            v, wt_ref[...], (((1,), (0,)), ((), ())),
            preferred_element_type=jnp.float32,
        )
        + b_ref[...]
    )
    s = sir[0:B]
    ii = sir[B:2 * B]
    row = i * B + jax.lax.broadcasted_iota(jnp.int32, (B, 1), 0)
    mask = (row < k).astype(jnp.float32)
    beta = x3_ref[0, :, 0:1]
    gamma = x3_ref[0, :, 1:2]
    ds = -beta * (ii * mask * s)
    dr = gamma * ii
    out_ref[0] = ds
    out_ref[1] = -ds - dr
    out_ref[2] = dr
    out_ref[3] = jnp.zeros_like(ds)


def kernel(t, x, W, b):
    del t
    n = x.shape[1]
    block_rows = 1000
    g = x[3, :, 2]
    gpad = jnp.pad(g, (0, _GP_ROWS * 128 - n)).reshape(_GP_ROWS, 128)
    wt = W.T
    b2 = b.reshape(1, _H)
    out = pl.pallas_call(
        functools.partial(_odefunc_body, block_rows=block_rows),
        grid=(n // block_rows,),
        in_specs=[
            pl.BlockSpec((_GP_ROWS, 128), lambda i: (0, 0)),
            pl.BlockSpec((2, block_rows, _H), lambda i: (0, i, 0)),
            pl.BlockSpec((1, block_rows, _H), lambda i: (3, i, 0)),
            pl.BlockSpec((_H, _H), lambda i: (0, 0)),
            pl.BlockSpec((1, _H), lambda i: (0, 0)),
        ],
        out_specs=pl.BlockSpec((4, block_rows, _H), lambda i: (0, i, 0)),
        out_shape=jax.ShapeDtypeStruct((4, n, _H), jnp.float32),
    )(gpad, x, x, wt, b2)
    return out


# speedup vs baseline: 2.5595x; 1.1693x over previous
import functools

import jax
import jax.numpy as jnp
from jax.experimental import pallas as pl

_N = 50000
_H = 128
_GP_ROWS = 400


def _odefunc_body(g_ref, si_ref, x3_ref, wt_ref, b_ref, out_ref, *, block_rows):
    i = pl.program_id(0)
    B = block_rows
    k = jnp.sum((g_ref[...] != 0.0).astype(jnp.int32))
    v = si_ref[...].reshape(2 * B, _H)
    sir = jax.nn.sigmoid(
        jax.lax.dot_general(
            v, wt_ref[...], (((1,), (0,)), ((), ())),
            preferred_element_type=jnp.float32,
        )
        + b_ref[...]
    )
    s = sir[0:B]
    ii = sir[B:2 * B]
    row = i * B + jax.lax.broadcasted_iota(jnp.int32, (B, 1), 0)
    mask = (row < k).astype(jnp.float32)
    beta = x3_ref[0, :, 0:1]
    gamma = x3_ref[0, :, 1:2]
    ds = -beta * (ii * mask * s)
    dr = gamma * ii
    out_ref[0] = ds
    out_ref[1] = -ds - dr
    out_ref[2] = dr
    out_ref[3] = jnp.zeros_like(ds)


def kernel(t, x, W, b):
    del t
    n = x.shape[1]
    block_rows = 2000
    g = x[3, :, 2]
    gpad = jnp.pad(g, (0, _GP_ROWS * 128 - n)).reshape(_GP_ROWS, 128)
    wt = W.T
    b2 = b.reshape(1, _H)
    out = pl.pallas_call(
        functools.partial(_odefunc_body, block_rows=block_rows),
        grid=(n // block_rows,),
        in_specs=[
            pl.BlockSpec((_GP_ROWS, 128), lambda i: (0, 0)),
            pl.BlockSpec((2, block_rows, _H), lambda i: (0, i, 0)),
            pl.BlockSpec((1, block_rows, _H), lambda i: (3, i, 0)),
            pl.BlockSpec((_H, _H), lambda i: (0, 0)),
            pl.BlockSpec((1, _H), lambda i: (0, 0)),
        ],
        out_specs=pl.BlockSpec((4, block_rows, _H), lambda i: (0, i, 0)),
        out_shape=jax.ShapeDtypeStruct((4, n, _H), jnp.float32),
    )(gpad, x, x, wt, b2)
    return out


# speedup vs baseline: 2.6568x; 1.0380x over previous
import functools

import jax
import jax.numpy as jnp
from jax.experimental import pallas as pl

_N = 50000
_H = 128
_GP_ROWS = 400


def _odefunc_body(g_ref, si_ref, x3_ref, wt_ref, b_ref, out_ref, *, block_rows):
    i = pl.program_id(0)
    B = block_rows
    k = jnp.sum((g_ref[...] != 0.0).astype(jnp.int32))
    v = si_ref[...].reshape(2 * B, _H)
    sir = jax.nn.sigmoid(
        jax.lax.dot_general(
            v, wt_ref[...], (((1,), (0,)), ((), ())),
            preferred_element_type=jnp.float32,
        )
        + b_ref[...]
    )
    s = sir[0:B]
    ii = sir[B:2 * B]
    row = i * B + jax.lax.broadcasted_iota(jnp.int32, (B, 1), 0)
    mask = (row < k).astype(jnp.float32)
    beta = x3_ref[0, :, 0:1]
    gamma = x3_ref[0, :, 1:2]
    ds = -beta * (ii * mask * s)
    dr = gamma * ii
    out_ref[0] = ds
    out_ref[1] = -ds - dr
    out_ref[2] = dr
    out_ref[3] = jnp.zeros_like(ds)


def kernel(t, x, W, b):
    del t
    n = x.shape[1]
    block_rows = 5000
    g = x[3, :, 2]
    gpad = jnp.pad(g, (0, _GP_ROWS * 128 - n)).reshape(_GP_ROWS, 128)
    wt = W.T
    b2 = b.reshape(1, _H)
    out = pl.pallas_call(
        functools.partial(_odefunc_body, block_rows=block_rows),
        grid=(n // block_rows,),
        in_specs=[
            pl.BlockSpec((_GP_ROWS, 128), lambda i: (0, 0)),
            pl.BlockSpec((2, block_rows, _H), lambda i: (0, i, 0)),
            pl.BlockSpec((1, block_rows, _H), lambda i: (3, i, 0)),
            pl.BlockSpec((_H, _H), lambda i: (0, 0)),
            pl.BlockSpec((1, _H), lambda i: (0, 0)),
        ],
        out_specs=pl.BlockSpec((4, block_rows, _H), lambda i: (0, i, 0)),
        out_shape=jax.ShapeDtypeStruct((4, n, _H), jnp.float32),
    )(gpad, x, x, wt, b2)
    return out
